# baseline (speedup 1.0000x reference)
import functools

import jax
import jax.numpy as jnp
from jax import lax
from jax.experimental import pallas as pl
from jax.experimental.pallas import tpu as pltpu
from jax.experimental.pallas import tpu_sc as plsc

H = 1024
E = 16
I = 512
N_GROUP = 4
GSIZE = E // N_GROUP
T = 2048
TOP_K = 2
NA = T * TOP_K
RT = 128
N_PAD = NA + E * RT
N_TILES = N_PAD // RT
CHUNK = 256

NC, NS = 2, 16
NW = NC * NS
PER_W = NA // NW
SC_CH = 64


def _route_kernel(x_ref, gatew_ref, pos_ref, w_ref, texp_ref):
    logits = lax.dot_general(x_ref[...], gatew_ref[...],
                             (((1,), (1,)), ((), ())),
                             preferred_element_type=jnp.float32)
    m = jnp.max(logits, axis=-1, keepdims=True)
    p = jnp.exp(logits - m)
    s = p / jnp.sum(p, axis=-1, keepdims=True)

    lane = lax.broadcasted_iota(jnp.int32, (T, E), 1)
    grp = lane // GSIZE
    gmax = jnp.zeros_like(s)
    for g in range(N_GROUP):
        mg = jnp.max(jnp.where(grp == g, s, -1.0), axis=-1, keepdims=True)
        gmax = jnp.where(grp == g, mg, gmax)
    vg1 = jnp.max(gmax, axis=-1, keepdims=True)
    l1 = jnp.min(jnp.where(gmax == vg1, lane, E), axis=-1, keepdims=True)
    g1 = l1 // GSIZE
    gmax2 = jnp.where(grp == g1, -1.0, gmax)
    vg2 = jnp.max(gmax2, axis=-1, keepdims=True)
    l2 = jnp.min(jnp.where(gmax2 == vg2, lane, E), axis=-1, keepdims=True)
    g2 = l2 // GSIZE
    ms = jnp.where((grp == g1) | (grp == g2), s, 0.0)
    v1 = jnp.max(ms, axis=-1, keepdims=True)
    i1 = jnp.min(jnp.where(ms == v1, lane, E), axis=-1, keepdims=True)
    ms2 = jnp.where(lane == i1, -1.0, ms)
    v2 = jnp.max(ms2, axis=-1, keepdims=True)
    i2 = jnp.min(jnp.where(ms2 == v2, lane, E), axis=-1, keepdims=True)
    denom = v1 + v2 + 1e-20
    w_ref[...] = jnp.concatenate([v1 / denom, v2 / denom], axis=1)

    amat = ((lane == i1) | (lane == i2)).astype(jnp.float32)
    ri = lax.broadcasted_iota(jnp.int32, (CHUNK, CHUNK), 0)
    ci = lax.broadcasted_iota(jnp.int32, (CHUNK, CHUNK), 1)
    ltri = (ci < ri).astype(jnp.float32)
    base = jnp.zeros((1, E), jnp.float32)
    pchunks = []
    for c in range(T // CHUNK):
        ac = amat[c * CHUNK:(c + 1) * CHUNK]
        pc = lax.dot_general(ltri, ac, (((1,), (0,)), ((), ())),
                             preferred_element_type=jnp.float32) + base
        base = base + jnp.sum(ac, axis=0, keepdims=True)
        pchunks.append(pc)
    prefix = jnp.concatenate(pchunks, axis=0)
    count = base
    padded = jnp.floor((count + (RT - 1)) * (1.0 / RT)) * RT
    ri16 = lax.broadcasted_iota(jnp.int32, (E, E), 0)
    ci16 = lax.broadcasted_iota(jnp.int32, (E, E), 1)
    mtri = (ci16 < ri16).astype(jnp.float32)
    pad_off = lax.dot_general(padded, mtri, (((1,), (1,)), ((), ())),
                              preferred_element_type=jnp.float32)
    pad_end = pad_off + padded

    pos_full = pad_off + prefix
    pos1 = jnp.sum(jnp.where(lane == i1, pos_full, 0.0), axis=-1,
                   keepdims=True)
    pos2 = jnp.sum(jnp.where(lane == i2, pos_full, 0.0), axis=-1,
                   keepdims=True)
    pos_ref[...] = jnp.concatenate([pos1, pos2], axis=1).astype(jnp.int32)

    tid = (lax.broadcasted_iota(jnp.int32, (N_TILES, E), 0)
           * RT).astype(jnp.float32)
    cmp = (tid >= pad_end).astype(jnp.float32)
    texp = jnp.sum(cmp, axis=1, keepdims=True)
    texp_ref[...] = jnp.minimum(texp, E - 1).astype(jnp.int32)


def _route(x, gate_w):
    return pl.pallas_call(
        _route_kernel,
        out_shape=[
            jax.ShapeDtypeStruct((T, TOP_K), jnp.int32),
            jax.ShapeDtypeStruct((T, TOP_K), jnp.float32),
            jax.ShapeDtypeStruct((N_TILES, 1), jnp.int32),
        ],
    )(x, gate_w)


def _sc_scatter(x, pos_flat):
    mesh = plsc.VectorSubcoreMesh(core_axis_name="c", subcore_axis_name="s")

    @functools.partial(
        pl.kernel, mesh=mesh,
        out_type=jax.ShapeDtypeStruct((N_PAD, H), jnp.float32),
        scratch_types=[
            pltpu.VMEM((SC_CH,), jnp.int32),
            pltpu.VMEM((SC_CH, H), jnp.float32),
            pltpu.SemaphoreType.DMA,
        ],
    )
    def k(x_hbm, idx_hbm, o_hbm, idx_v, rows_v, sem):
        wid = lax.axis_index("s") * NC + lax.axis_index("c")
        jbase = wid * PER_W

        @pl.loop(0, PER_W // SC_CH)
        def _(c):
            base = jbase + c * SC_CH
            tbase = lax.rem(base, T)
            pltpu.sync_copy(idx_hbm.at[pl.ds(base, SC_CH)], idx_v)
            pltpu.sync_copy(x_hbm.at[pl.ds(tbase, SC_CH)], rows_v)
            pltpu.async_copy(rows_v, o_hbm.at[idx_v], sem).wait()

    return k(x, pos_flat)


def _sc_gather(y, pos_flat):
    mesh = plsc.VectorSubcoreMesh(core_axis_name="c", subcore_axis_name="s")

    @functools.partial(
        pl.kernel, mesh=mesh,
        out_type=jax.ShapeDtypeStruct((NA, H), jnp.float32),
        scratch_types=[
            pltpu.VMEM((SC_CH,), jnp.int32),
            pltpu.VMEM((SC_CH, H), jnp.float32),
            pltpu.SemaphoreType.DMA,
        ],
    )
    def k(y_hbm, idx_hbm, o_hbm, idx_v, rows_v, sem):
        wid = lax.axis_index("s") * NC + lax.axis_index("c")
        jbase = wid * PER_W

        @pl.loop(0, PER_W // SC_CH)
        def _(c):
            base = jbase + c * SC_CH
            pltpu.sync_copy(idx_hbm.at[pl.ds(base, SC_CH)], idx_v)
            pltpu.async_copy(y_hbm.at[idx_v], rows_v, sem).wait()
            pltpu.sync_copy(rows_v, o_hbm.at[pl.ds(base, SC_CH)])

    return k(y, pos_flat)


def _mlp(xb, g_ref, u_ref, d_ref):
    wg = g_ref[0].astype(jnp.bfloat16)
    wu = u_ref[0].astype(jnp.bfloat16)
    wd = d_ref[0].astype(jnp.bfloat16)
    hg = lax.dot_general(xb, wg, (((1,), (1,)), ((), ())),
                         preferred_element_type=jnp.float32)
    hu = lax.dot_general(xb, wu, (((1,), (1,)), ((), ())),
                         preferred_element_type=jnp.float32)
    h = (jax.nn.silu(hg) * hu).astype(jnp.bfloat16)
    return lax.dot_general(h, wd, (((1,), (1,)), ((), ())),
                           preferred_element_type=jnp.float32)


def _grouped_kernel(texp_ref, x_ref, wg_ref, wu_ref, wd_ref, o_ref):
    del texp_ref
    xb = x_ref[...].astype(jnp.bfloat16)
    o_ref[...] = _mlp(xb, wg_ref, wu_ref, wd_ref)


def _grouped_mlp(texp, x_sorted, w_gate, w_up, w_down):
    grid_spec = pltpu.PrefetchScalarGridSpec(
        num_scalar_prefetch=1,
        grid=(N_TILES,),
        in_specs=[
            pl.BlockSpec((RT, H), lambda i, texp: (i, 0)),
            pl.BlockSpec((1, I, H), lambda i, texp: (texp[i], 0, 0)),
            pl.BlockSpec((1, I, H), lambda i, texp: (texp[i], 0, 0)),
            pl.BlockSpec((1, H, I), lambda i, texp: (texp[i], 0, 0)),
        ],
        out_specs=pl.BlockSpec((RT, H), lambda i, texp: (i, 0)),
    )
    return pl.pallas_call(
        _grouped_kernel,
        grid_spec=grid_spec,
        out_shape=jax.ShapeDtypeStruct((N_PAD, H), jnp.float32),
        compiler_params=pltpu.CompilerParams(
            dimension_semantics=("arbitrary",)),
    )(texp, x_sorted, w_gate, w_up, w_down)


SH_TB = 1024


def _shared_kernel(x_ref, wsg_ref, wsu_ref, wsd_ref, o_ref):
    e = pl.program_id(1)

    @pl.when(e == 0)
    def _():
        o_ref[...] = jnp.zeros_like(o_ref)

    xb = x_ref[...].astype(jnp.bfloat16)
    o_ref[...] += _mlp(xb, wsg_ref, wsu_ref, wsd_ref)


def _shared_mlp(x, wsg, wsu, wsd):
    return pl.pallas_call(
        _shared_kernel,
        grid=(T // SH_TB, 2),
        in_specs=[
            pl.BlockSpec((SH_TB, H), lambda h, e: (h, 0)),
            pl.BlockSpec((1, I, H), lambda h, e: (e, 0, 0)),
            pl.BlockSpec((1, I, H), lambda h, e: (e, 0, 0)),
            pl.BlockSpec((1, H, I), lambda h, e: (e, 0, 0)),
        ],
        out_specs=pl.BlockSpec((SH_TB, H), lambda h, e: (h, 0)),
        out_shape=jax.ShapeDtypeStruct((T, H), jnp.float32),
        compiler_params=pltpu.CompilerParams(
            dimension_semantics=("parallel", "arbitrary")),
    )(x, wsg, wsu, wsd)


def _combine_kernel(y1_ref, y2_ref, w_ref, sh_ref, o_ref):
    w = w_ref[...]
    o_ref[...] = (w[:, 0:1] * y1_ref[...] + w[:, 1:2] * y2_ref[...]
                  + sh_ref[...])


def _combine(y01, w, shared):
    return pl.pallas_call(
        _combine_kernel,
        grid=(T // SH_TB,),
        in_specs=[
            pl.BlockSpec((SH_TB, H), lambda h: (h, 0)),
            pl.BlockSpec((SH_TB, H), lambda h: (h + T // SH_TB, 0)),
            pl.BlockSpec((SH_TB, TOP_K), lambda h: (h, 0)),
            pl.BlockSpec((SH_TB, H), lambda h: (h, 0)),
        ],
        out_specs=pl.BlockSpec((SH_TB, H), lambda h: (h, 0)),
        out_shape=jax.ShapeDtypeStruct((T, H), jnp.float32),
        compiler_params=pltpu.CompilerParams(
            dimension_semantics=("parallel",)),
    )(y01, y01, w, shared)


def kernel(hidden_states, gate_w, w_gate, w_up, w_down,
           ws_gate, ws_up, ws_down):
    x = hidden_states
    pos, w01, texp = _route(x, gate_w)
    pos_flat = pos.T.reshape(NA)
    x_sorted = _sc_scatter(x, pos_flat)
    wsg = ws_gate.reshape(2, I, H)
    wsu = ws_up.reshape(2, I, H)
    wsd = jnp.stack([ws_down[:, :I], ws_down[:, I:]])
    shared = _shared_mlp(x, wsg, wsu, wsd)
    y = _grouped_mlp(texp.reshape(N_TILES), x_sorted, w_gate, w_up, w_down)
    y01 = _sc_gather(y, pos_flat)
    return _combine(y01, w01, shared)
